# baseline (speedup 1.0000x reference)
import functools

import jax
import jax.numpy as jnp
from jax import lax
from jax.experimental import pallas as pl
from jax.experimental.pallas import tpu as pltpu
from jax.experimental.pallas import tpu_sc as plsc

T = 64
H = 1024
F = 4096
E = 8
TOP_K = 2

BF = 1024
NF = F // BF

L = 16
NGROUPS = T // L


def _logits_kernel(x_ref, gw_ref, out_ref):
    out_ref[...] = jax.lax.dot_general(
        gw_ref[...], x_ref[...], (((0,), (1,)), ((), ())),
        preferred_element_type=jnp.float32,
    )


def _router_sc_kernel(logits_hbm, comb_hbm, lv, cv):
    wid = lax.axis_index("s") * 2 + lax.axis_index("c")

    @pl.when(wid == 0)
    def _():
        pltpu.sync_copy(logits_hbm, lv)
        for g in range(NGROUPS):
            t0 = g * L
            rows = [lv[pl.ds(e * T + t0, L)] for e in range(E)]
            v1 = rows[0]
            for e in range(1, E):
                v1 = jnp.maximum(v1, rows[e])
            big = jnp.full((L,), E, jnp.int32)
            i1 = big
            for e in range(E - 1, -1, -1):
                i1 = jnp.where(rows[e] == v1, jnp.full((L,), e, jnp.int32), i1)
            neg = jnp.full((L,), -jnp.inf, jnp.float32)
            rwm = [jnp.where(i1 == e, neg, rows[e]) for e in range(E)]
            v2 = rwm[0]
            for e in range(1, E):
                v2 = jnp.maximum(v2, rwm[e])
            i2 = big
            for e in range(E - 1, -1, -1):
                i2 = jnp.where(rwm[e] == v2, jnp.full((L,), e, jnp.int32), i2)
            zero = jnp.zeros((L,), jnp.float32)
            kept = [jnp.where((i1 == e) | (i2 == e),
                              jnp.exp(rows[e] - v1), zero)
                    for e in range(E)]
            s2 = kept[0]
            for e in range(1, E):
                s2 = s2 + kept[e]
            for e in range(E):
                cv[pl.ds(e * T + t0, L)] = kept[e] / s2
        pltpu.sync_copy(cv, comb_hbm)


_router_sc = functools.partial(
    pl.kernel,
    mesh=plsc.VectorSubcoreMesh(core_axis_name="c", subcore_axis_name="s"),
    out_type=jax.ShapeDtypeStruct((E * T,), jnp.float32),
    scratch_types=[
        pltpu.VMEM((E * T,), jnp.float32),
        pltpu.VMEM((E * T,), jnp.float32),
    ],
)(_router_sc_kernel)


BH = 512
NH = H // BH
NS = NF + NH


def _moe_kernel(x_ref, comb_ref, w1_ref, w3_ref, w2_ref, out_ref, act_ref):
    e = pl.program_id(0)
    s = pl.program_id(1)

    @pl.when((e == 0) & (s == 0))
    def _init():
        out_ref[...] = jnp.zeros_like(out_ref)

    @pl.when(s < NF)
    def _up():
        x = x_ref[...]
        w1b = w1_ref[0]
        w3b = w3_ref[0]
        g = jax.lax.dot_general(
            x, w1b, (((1,), (1,)), ((), ())),
            preferred_element_type=jnp.float32)
        u = jax.lax.dot_general(
            x, w3b, (((1,), (1,)), ((), ())),
            preferred_element_type=jnp.float32)
        eidx = jax.lax.broadcasted_iota(jnp.int32, (E, T), 0)
        c = jnp.sum(jnp.where(eidx == e, comb_ref[...], 0.0), axis=0)
        act_ref[:, pl.ds(s * BF, BF)] = (g * jax.nn.sigmoid(g)) * u * c[:, None]

    @pl.when(s >= NF)
    def _down():
        h = s - NF
        w2b = w2_ref[0]
        out_ref[:, pl.ds(h * BH, BH)] += jax.lax.dot_general(
            act_ref[...], w2b, (((1,), (1,)), ((), ())),
            preferred_element_type=jnp.float32)


@jax.jit
def kernel(hidden_states, gate_w, w1, w3, w2):
    logits_t = pl.pallas_call(
        _logits_kernel,
        out_shape=jax.ShapeDtypeStruct((E, T), jnp.float32),
    )(hidden_states, gate_w)
    comb_f = _router_sc(logits_t.reshape(E * T))
    return pl.pallas_call(
        _moe_kernel,
        grid=(E, NS),
        in_specs=[
            pl.BlockSpec((T, H), lambda e, s: (0, 0)),
            pl.BlockSpec((E, T), lambda e, s: (0, 0)),
            pl.BlockSpec((1, BF, H), lambda e, s: (e, jnp.minimum(s, NF - 1), 0)),
            pl.BlockSpec((1, BF, H), lambda e, s: (e, jnp.minimum(s, NF - 1), 0)),
            pl.BlockSpec((1, BH, F), lambda e, s: (e, jnp.maximum(s - NF, 0), 0)),
        ],
        out_specs=pl.BlockSpec((T, H), lambda e, s: (0, 0)),
        out_shape=jax.ShapeDtypeStruct((T, H), jnp.float32),
        scratch_shapes=[pltpu.VMEM((T, F), jnp.float32)],
        compiler_params=pltpu.CompilerParams(
            dimension_semantics=("arbitrary", "arbitrary"),
        ),
    )(hidden_states, comb_f.reshape(E, T), w1, w3, w2)


# speedup vs baseline: 1.1508x; 1.1508x over previous
import functools

import jax
import jax.numpy as jnp
from jax import lax
from jax.experimental import pallas as pl
from jax.experimental.pallas import tpu as pltpu
from jax.experimental.pallas import tpu_sc as plsc

T = 64
H = 1024
F = 4096
E = 8
TOP_K = 2

BF = 1024
NF = F // BF

L = 16
NGROUPS = T // L


def _logits_kernel(x_ref, gw_ref, out_ref):
    out_ref[...] = jax.lax.dot_general(
        gw_ref[...], x_ref[...], (((0,), (1,)), ((), ())),
        preferred_element_type=jnp.float32,
    )


def _router_sc_kernel(logits_hbm, comb_hbm, lv, cv):
    wid = lax.axis_index("s") * 2 + lax.axis_index("c")

    @pl.when(wid == 0)
    def _():
        pltpu.sync_copy(logits_hbm, lv)
        for g in range(NGROUPS):
            t0 = g * L
            rows = [lv[pl.ds(e * T + t0, L)] for e in range(E)]
            v1 = rows[0]
            for e in range(1, E):
                v1 = jnp.maximum(v1, rows[e])
            big = jnp.full((L,), E, jnp.int32)
            i1 = big
            for e in range(E - 1, -1, -1):
                i1 = jnp.where(rows[e] == v1, jnp.full((L,), e, jnp.int32), i1)
            neg = jnp.full((L,), -jnp.inf, jnp.float32)
            rwm = [jnp.where(i1 == e, neg, rows[e]) for e in range(E)]
            v2 = rwm[0]
            for e in range(1, E):
                v2 = jnp.maximum(v2, rwm[e])
            i2 = big
            for e in range(E - 1, -1, -1):
                i2 = jnp.where(rwm[e] == v2, jnp.full((L,), e, jnp.int32), i2)
            zero = jnp.zeros((L,), jnp.float32)
            kept = [jnp.where((i1 == e) | (i2 == e),
                              jnp.exp(rows[e] - v1), zero)
                    for e in range(E)]
            s2 = kept[0]
            for e in range(1, E):
                s2 = s2 + kept[e]
            for e in range(E):
                cv[pl.ds(e * T + t0, L)] = kept[e] / s2
        pltpu.sync_copy(cv, comb_hbm)


_router_sc = functools.partial(
    pl.kernel,
    mesh=plsc.VectorSubcoreMesh(core_axis_name="c", subcore_axis_name="s"),
    out_type=jax.ShapeDtypeStruct((E * T,), jnp.float32),
    scratch_types=[
        pltpu.VMEM((E * T,), jnp.float32),
        pltpu.VMEM((E * T,), jnp.float32),
    ],
)(_router_sc_kernel)


def _moe_kernel(x_ref, w1_ref, w3_ref, w2_ref, eo_ref):
    f = pl.program_id(1)

    x = x_ref[...]
    w1b = w1_ref[0]
    w3b = w3_ref[0]
    w2b = w2_ref[0]
    g = jax.lax.dot_general(
        x, w1b, (((1,), (1,)), ((), ())), preferred_element_type=jnp.float32)
    u = jax.lax.dot_general(
        x, w3b, (((1,), (1,)), ((), ())), preferred_element_type=jnp.float32)
    act = (g * jax.nn.sigmoid(g)) * u
    contrib = jax.lax.dot_general(
        act, w2b, (((1,), (1,)), ((), ())), preferred_element_type=jnp.float32)

    @pl.when(f == 0)
    def _first():
        eo_ref[0] = contrib

    @pl.when(f != 0)
    def _rest():
        eo_ref[0] += contrib


def _combine_kernel(comb_ref, eo_ref, out_ref):
    comb = comb_ref[...]
    acc = comb[0, :, None] * eo_ref[0]
    for e in range(1, E):
        acc += comb[e, :, None] * eo_ref[e]
    out_ref[...] = acc


@jax.jit
def kernel(hidden_states, gate_w, w1, w3, w2):
    logits_t = pl.pallas_call(
        _logits_kernel,
        out_shape=jax.ShapeDtypeStruct((E, T), jnp.float32),
    )(hidden_states, gate_w)
    comb_f = _router_sc(logits_t.reshape(E * T))
    expert_out = pl.pallas_call(
        _moe_kernel,
        grid=(E, NF),
        in_specs=[
            pl.BlockSpec((T, H), lambda e, f: (0, 0)),
            pl.BlockSpec((1, BF, H), lambda e, f: (e, f, 0)),
            pl.BlockSpec((1, BF, H), lambda e, f: (e, f, 0)),
            pl.BlockSpec((1, H, BF), lambda e, f: (e, 0, f)),
        ],
        out_specs=pl.BlockSpec((1, T, H), lambda e, f: (e, 0, 0)),
        out_shape=jax.ShapeDtypeStruct((E, T, H), jnp.float32),
        compiler_params=pltpu.CompilerParams(
            dimension_semantics=("arbitrary", "arbitrary"),
        ),
    )(hidden_states, w1, w3, w2)
    return pl.pallas_call(
        _combine_kernel,
        out_shape=jax.ShapeDtypeStruct((T, H), jnp.float32),
    )(comb_f.reshape(E, T), expert_out)


# speedup vs baseline: 1.1571x; 1.0055x over previous
import functools

import jax
import jax.numpy as jnp
from jax import lax
from jax.experimental import pallas as pl
from jax.experimental.pallas import tpu as pltpu
from jax.experimental.pallas import tpu_sc as plsc

T = 64
H = 1024
F = 4096
E = 8
TOP_K = 2

BF = 1024
NF = F // BF

L = 16
NGROUPS = T // L


def _logits_kernel(x_ref, gw_ref, out_ref):
    out_ref[...] = jax.lax.dot_general(
        gw_ref[...], x_ref[...], (((0,), (1,)), ((), ())),
        preferred_element_type=jnp.float32,
    )


def _router_sc_kernel(logits_hbm, comb_hbm, lv, cv):
    wid = lax.axis_index("s") * 2 + lax.axis_index("c")

    @pl.when(wid == 0)
    def _():
        pltpu.sync_copy(logits_hbm, lv)
        for g in range(NGROUPS):
            t0 = g * L
            rows = [lv[pl.ds(e * T + t0, L)] for e in range(E)]
            v1 = rows[0]
            for e in range(1, E):
                v1 = jnp.maximum(v1, rows[e])
            big = jnp.full((L,), E, jnp.int32)
            i1 = big
            for e in range(E - 1, -1, -1):
                i1 = jnp.where(rows[e] == v1, jnp.full((L,), e, jnp.int32), i1)
            neg = jnp.full((L,), -jnp.inf, jnp.float32)
            rwm = [jnp.where(i1 == e, neg, rows[e]) for e in range(E)]
            v2 = rwm[0]
            for e in range(1, E):
                v2 = jnp.maximum(v2, rwm[e])
            i2 = big
            for e in range(E - 1, -1, -1):
                i2 = jnp.where(rwm[e] == v2, jnp.full((L,), e, jnp.int32), i2)
            zero = jnp.zeros((L,), jnp.float32)
            kept = [jnp.where((i1 == e) | (i2 == e),
                              jnp.exp(rows[e] - v1), zero)
                    for e in range(E)]
            s2 = kept[0]
            for e in range(1, E):
                s2 = s2 + kept[e]
            for e in range(E):
                cv[pl.ds(e * T + t0, L)] = kept[e] / s2
        pltpu.sync_copy(cv, comb_hbm)


_router_sc = functools.partial(
    pl.kernel,
    mesh=plsc.VectorSubcoreMesh(core_axis_name="c", subcore_axis_name="s"),
    out_type=jax.ShapeDtypeStruct((E * T,), jnp.float32),
    scratch_types=[
        pltpu.VMEM((E * T,), jnp.float32),
        pltpu.VMEM((E * T,), jnp.float32),
    ],
)(_router_sc_kernel)


def _moe_kernel(x_ref, w1_ref, w3_ref, w2_ref, eo_ref):
    f = pl.program_id(1)

    x = x_ref[...]
    w1b = w1_ref[0]
    w3b = w3_ref[0]
    w2b = w2_ref[0]
    g = jax.lax.dot_general(
        x, w1b, (((1,), (1,)), ((), ())), preferred_element_type=jnp.float32)
    u = jax.lax.dot_general(
        x, w3b, (((1,), (1,)), ((), ())), preferred_element_type=jnp.float32)
    act = (g * jax.nn.sigmoid(g)) * u
    contrib = jax.lax.dot_general(
        act, w2b, (((1,), (1,)), ((), ())), preferred_element_type=jnp.float32)

    @pl.when(f == 0)
    def _first():
        eo_ref[0] = contrib

    @pl.when(f != 0)
    def _rest():
        eo_ref[0] += contrib


def _combine_kernel(comb_ref, eo_ref, out_ref):
    comb = comb_ref[...]
    acc = comb[0, :, None] * eo_ref[0]
    for e in range(1, E):
        acc += comb[e, :, None] * eo_ref[e]
    out_ref[...] = acc


@jax.jit
def kernel(hidden_states, gate_w, w1, w3, w2):
    logits_t = pl.pallas_call(
        _logits_kernel,
        out_shape=jax.ShapeDtypeStruct((E, T), jnp.float32),
    )(hidden_states, gate_w)
    comb_f = _router_sc(logits_t.reshape(E * T))
    expert_out = pl.pallas_call(
        _moe_kernel,
        grid=(E, NF),
        in_specs=[
            pl.BlockSpec((T, H), lambda e, f: (0, 0)),
            pl.BlockSpec((1, BF, H), lambda e, f: (e, f, 0)),
            pl.BlockSpec((1, BF, H), lambda e, f: (e, f, 0)),
            pl.BlockSpec((1, H, BF), lambda e, f: (e, 0, f)),
        ],
        out_specs=pl.BlockSpec((1, T, H), lambda e, f: (e, 0, 0)),
        out_shape=jax.ShapeDtypeStruct((E, T, H), jnp.float32),
        compiler_params=pltpu.CompilerParams(
            dimension_semantics=("parallel", "arbitrary"),
        ),
    )(hidden_states, w1, w3, w2)
    return pl.pallas_call(
        _combine_kernel,
        out_shape=jax.ShapeDtypeStruct((T, H), jnp.float32),
    )(comb_f.reshape(E, T), expert_out)


# speedup vs baseline: 1.1590x; 1.0017x over previous
import functools

import jax
import jax.numpy as jnp
from jax import lax
from jax.experimental import pallas as pl
from jax.experimental.pallas import tpu as pltpu
from jax.experimental.pallas import tpu_sc as plsc

T = 64
H = 1024
F = 4096
E = 8
TOP_K = 2

BF = 1024
NF = F // BF

L = 16
NGROUPS = T // L


def _logits_kernel(x_ref, gw_ref, out_ref):
    out_ref[...] = jax.lax.dot_general(
        gw_ref[...], x_ref[...], (((0,), (1,)), ((), ())),
        preferred_element_type=jnp.float32,
    )


def _router_sc_kernel(logits_hbm, comb_hbm, lv, cv):
    wid = lax.axis_index("s") * 2 + lax.axis_index("c")

    @pl.when(wid == 0)
    def _():
        pltpu.sync_copy(logits_hbm, lv)
        for g in range(NGROUPS):
            t0 = g * L
            rows = [lv[pl.ds(e * T + t0, L)] for e in range(E)]
            v1 = rows[0]
            for e in range(1, E):
                v1 = jnp.maximum(v1, rows[e])
            big = jnp.full((L,), E, jnp.int32)
            i1 = big
            for e in range(E - 1, -1, -1):
                i1 = jnp.where(rows[e] == v1, jnp.full((L,), e, jnp.int32), i1)
            neg = jnp.full((L,), -jnp.inf, jnp.float32)
            rwm = [jnp.where(i1 == e, neg, rows[e]) for e in range(E)]
            v2 = rwm[0]
            for e in range(1, E):
                v2 = jnp.maximum(v2, rwm[e])
            i2 = big
            for e in range(E - 1, -1, -1):
                i2 = jnp.where(rwm[e] == v2, jnp.full((L,), e, jnp.int32), i2)
            zero = jnp.zeros((L,), jnp.float32)
            kept = [jnp.where((i1 == e) | (i2 == e),
                              jnp.exp(rows[e] - v1), zero)
                    for e in range(E)]
            s2 = kept[0]
            for e in range(1, E):
                s2 = s2 + kept[e]
            for e in range(E):
                cv[pl.ds(e * T + t0, L)] = kept[e] / s2
        pltpu.sync_copy(cv, comb_hbm)


_router_sc = functools.partial(
    pl.kernel,
    mesh=plsc.VectorSubcoreMesh(core_axis_name="c", subcore_axis_name="s"),
    out_type=jax.ShapeDtypeStruct((E * T,), jnp.float32),
    scratch_types=[
        pltpu.VMEM((E * T,), jnp.float32),
        pltpu.VMEM((E * T,), jnp.float32),
    ],
)(_router_sc_kernel)


def _moe_kernel(x_ref, comb_ref, w1_ref, w3_ref, w2_ref, out_ref):
    e = pl.program_id(0)
    f = pl.program_id(1)

    @pl.when((e == 0) & (f == 0))
    def _init():
        out_ref[...] = jnp.zeros_like(out_ref)

    x = x_ref[...]
    w1b = w1_ref[0]
    w3b = w3_ref[0]
    w2b = w2_ref[0]
    g = jax.lax.dot_general(
        x, w1b, (((1,), (1,)), ((), ())), preferred_element_type=jnp.float32)
    u = jax.lax.dot_general(
        x, w3b, (((1,), (1,)), ((), ())), preferred_element_type=jnp.float32)
    eidx = jax.lax.broadcasted_iota(jnp.int32, (E, T), 0)
    c = jnp.sum(jnp.where(eidx == e, comb_ref[...], 0.0), axis=0)
    act = (g * jax.nn.sigmoid(g)) * u * c[:, None]
    out_ref[...] += jax.lax.dot_general(
        act, w2b, (((1,), (1,)), ((), ())), preferred_element_type=jnp.float32)


@jax.jit
def kernel(hidden_states, gate_w, w1, w3, w2):
    logits_t = pl.pallas_call(
        _logits_kernel,
        out_shape=jax.ShapeDtypeStruct((E, T), jnp.float32),
    )(hidden_states, gate_w)
    comb_f = _router_sc(logits_t.reshape(E * T))
    return pl.pallas_call(
        _moe_kernel,
        grid=(E, NF),
        in_specs=[
            pl.BlockSpec((T, H), lambda e, f: (0, 0)),
            pl.BlockSpec((E, T), lambda e, f: (0, 0)),
            pl.BlockSpec((1, BF, H), lambda e, f: (e, f, 0)),
            pl.BlockSpec((1, BF, H), lambda e, f: (e, f, 0)),
            pl.BlockSpec((1, H, BF), lambda e, f: (e, 0, f)),
        ],
        out_specs=pl.BlockSpec((T, H), lambda e, f: (0, 0)),
        out_shape=jax.ShapeDtypeStruct((T, H), jnp.float32),
        compiler_params=pltpu.CompilerParams(
            dimension_semantics=("arbitrary", "arbitrary"),
        ),
    )(hidden_states, comb_f.reshape(E, T), w1, w3, w2)
